# baseline (speedup 1.0000x reference)
import functools

import jax
import jax.numpy as jnp
import numpy as np
from jax import lax
from jax.experimental import pallas as pl
from jax.experimental.pallas import tpu as pltpu
from jax.experimental.pallas import tpu_sc as plsc
from jax.sharding import Mesh, PartitionSpec as P

N_NODES = 10000
N_EDGES = 320000
D = 128
ND = 2
NC = 2
NS = 16
NW = NC * NS
CHUNK = 128
EPW = 5120
E_PAD = ND * NW * EPW
NCHUNK = EPW // CHUNK
IBLK = 8
NBLK = NCHUNK // IBLK
DEPTH = 2
NP = 10240
RPS = NP // NS
HND = NP // ND
CNT_W = 128
BN_SCALE = 1.0 / np.sqrt(1.0 + 1e-5)


def _seg_sum_body(h_hbm, src_hbm, dst_hbm, zrow_hbm, agg_out,
                  srcv, dstv, rows, sems, aggsh):
    c = lax.axis_index("c")
    s = lax.axis_index("s")
    wid = c * NS + s

    pltpu.sync_copy(zrow_hbm, aggsh.at[pl.ds(s * RPS, RPS)])
    plsc.subcore_barrier()

    def start_gather(ci, k):
        pltpu.async_copy(h_hbm.at[srcv.at[ci]], rows[k], sems[k])

    def wait_gather(ci, k):
        pltpu.make_async_copy(h_hbm.at[srcv.at[ci]], rows[k], sems[k]).wait()

    def scatter_add(ci, k):
        pltpu.sync_copy(rows[k], aggsh.at[dstv.at[ci]], add=True)

    @pl.loop(0, NBLK)
    def _(blk):
        pltpu.sync_copy(src_hbm.at[wid].at[pl.ds(blk * IBLK, IBLK)], srcv)
        pltpu.sync_copy(dst_hbm.at[wid].at[pl.ds(blk * IBLK, IBLK)], dstv)
        for k in range(DEPTH):
            start_gather(k, k)

        @pl.loop(0, IBLK - DEPTH, step=DEPTH)
        def _(ci):
            for k in range(DEPTH):
                wait_gather(ci + k, k)
                scatter_add(ci + k, k)
                start_gather(ci + k + DEPTH, k)

        for k in range(DEPTH):
            wait_gather(IBLK - DEPTH + k, k)
            scatter_add(IBLK - DEPTH + k, k)

    plsc.subcore_barrier()
    pltpu.sync_copy(aggsh.at[pl.ds(s * RPS, RPS)],
                    agg_out.at[c].at[pl.ds(s * RPS, RPS)])


_seg_sum = pl.kernel(
    _seg_sum_body,
    out_type=jax.ShapeDtypeStruct((NC, NP, D), jnp.float32),
    mesh=plsc.VectorSubcoreMesh(core_axis_name="c", subcore_axis_name="s"),
    scratch_types=[
        pltpu.VMEM((IBLK, CHUNK), jnp.int32),
        pltpu.VMEM((IBLK, CHUNK), jnp.int32),
        [pltpu.VMEM((CHUNK, D), jnp.float32) for _ in range(DEPTH)],
        [pltpu.SemaphoreType.DMA for _ in range(DEPTH)],
        pltpu.VMEM_SHARED((NP, D), jnp.float32),
    ],
    name="seg_sum",
)


def _cnt_body(dst_hbm, zcnt_hbm, ones_hbm, cnt_out, dstv, onesv, cntsh):
    c = lax.axis_index("c")
    s = lax.axis_index("s")
    wid = c * NS + s

    pltpu.sync_copy(zcnt_hbm, cntsh.at[pl.ds(s * RPS, RPS)])
    pltpu.sync_copy(ones_hbm, onesv)
    plsc.subcore_barrier()

    @pl.loop(0, NBLK)
    def _(blk):
        pltpu.sync_copy(dst_hbm.at[wid].at[pl.ds(blk * IBLK, IBLK)], dstv)

        @pl.loop(0, IBLK)
        def _(ci):
            pltpu.sync_copy(onesv, cntsh.at[dstv.at[ci]], add=True)

    plsc.subcore_barrier()
    pltpu.sync_copy(cntsh.at[pl.ds(s * RPS, RPS)],
                    cnt_out.at[c].at[pl.ds(s * RPS, RPS)])


_seg_cnt = pl.kernel(
    _cnt_body,
    out_type=jax.ShapeDtypeStruct((NC, NP, CNT_W), jnp.float32),
    mesh=plsc.VectorSubcoreMesh(core_axis_name="c", subcore_axis_name="s"),
    scratch_types=[
        pltpu.VMEM((IBLK, CHUNK), jnp.int32),
        pltpu.VMEM((CHUNK, CNT_W), jnp.float32),
        pltpu.VMEM_SHARED((NP, CNT_W), jnp.float32),
    ],
    name="seg_cnt",
)


def _tc_layer_body(relu, h_ref, pa_ref, inv_ref, wr_ref, b_ref,
                   wt_ref, g_ref, be_ref, o_ref):
    agg = (pa_ref[0] + pa_ref[1]) * inv_ref[:, 0:1]
    out = (jnp.dot(agg, wr_ref[...], preferred_element_type=jnp.float32)
           + b_ref[...]
           + jnp.dot(h_ref[...], wt_ref[...], preferred_element_type=jnp.float32))
    if relu:
        out = jnp.maximum(out * (g_ref[...] * BN_SCALE) + be_ref[...], 0.0)
    o_ref[...] = out


def _make_tc_layer(relu, rows=HND, block_rows=1280):
    grid = (rows // block_rows,)
    return pl.pallas_call(
        functools.partial(_tc_layer_body, relu),
        grid=grid,
        in_specs=[
            pl.BlockSpec((block_rows, D), lambda i: (i, 0)),
            pl.BlockSpec((ND, block_rows, D), lambda i: (0, i, 0)),
            pl.BlockSpec((block_rows, CNT_W), lambda i: (i, 0)),
            pl.BlockSpec((D, D), lambda i: (0, 0)),
            pl.BlockSpec((1, D), lambda i: (0, 0)),
            pl.BlockSpec((D, D), lambda i: (0, 0)),
            pl.BlockSpec((1, D), lambda i: (0, 0)),
            pl.BlockSpec((1, D), lambda i: (0, 0)),
        ],
        out_specs=pl.BlockSpec((block_rows, D), lambda i: (i, 0)),
        out_shape=jax.ShapeDtypeStruct((rows, D), jnp.float32),
        name="gconv_dense_relu" if relu else "gconv_dense",
    )


_tc_layer_relu = _make_tc_layer(True)
_tc_layer_plain = _make_tc_layer(False)


def _combine_body(pa_ref, o_ref):
    o_ref[...] = pa_ref[0] + pa_ref[1]


_tc_combine = pl.pallas_call(
    _combine_body,
    grid=(NP // 1280,),
    in_specs=[pl.BlockSpec((NC, 1280, D), lambda i: (0, i, 0))],
    out_specs=pl.BlockSpec((1280, D), lambda i: (i, 0)),
    out_shape=jax.ShapeDtypeStruct((NP, D), jnp.float32),
    name="combine_partials",
)


def _inv_body(c_ref, o_ref):
    o_ref[...] = 1.0 / jnp.maximum(c_ref[0] + c_ref[1], 1.0)


_tc_inv = pl.pallas_call(
    _inv_body,
    grid=(HND // 1280,),
    in_specs=[pl.BlockSpec((ND, 1280, CNT_W), lambda i: (0, i, 0))],
    out_specs=pl.BlockSpec((1280, CNT_W), lambda i: (i, 0)),
    out_shape=jax.ShapeDtypeStruct((HND, CNT_W), jnp.float32),
    name="inv_counts",
)


def _pipeline(hP, src, dst, zrow, ones, WrT1, b1, WtT1, g1, be1,
              WrT2, b2, WtT2, g2, be2, WrT3, b3, WtT3, dummy):
    d = lax.axis_index("d")
    src = src.reshape(NW, NCHUNK, CHUNK)
    dst = dst.reshape(NW, NCHUNK, CHUNK)

    def local_rows(a):
        return lax.dynamic_slice_in_dim(a, d * HND, HND, axis=0)

    def gather_planes(tot):
        g = lax.all_gather(tot, "d", axis=0, tiled=False)
        return lax.dynamic_slice_in_dim(g, d * HND, HND, axis=1)

    cnts = _seg_cnt(dst, zrow, ones)
    invL = _tc_inv(gather_planes(_tc_combine(cnts)))

    def layer(hP, tc, *w):
        agg = _seg_sum(hP, src, dst, zrow)
        pa = gather_planes(_tc_combine(agg))
        return tc(local_rows(hP), pa, invL, *w)

    h1 = layer(hP, _tc_layer_relu, WrT1, b1, WtT1, g1, be1)
    h1P = lax.all_gather(h1, "d", axis=0, tiled=True)
    h2 = layer(h1P, _tc_layer_relu, WrT2, b2, WtT2, g2, be2)
    h2P = lax.all_gather(h2, "d", axis=0, tiled=True)
    h3 = layer(h2P, _tc_layer_plain, WrT3, b3, WtT3, dummy, dummy)
    return lax.all_gather(h3, "d", axis=0, tiled=True)


def kernel(x, edge_index, W_rel1, b_rel1, W_root1, W_rel2, b_rel2, W_root2,
           W_rel3, b_rel3, W_root3, gamma1, beta1, gamma2, beta2):
    npad = E_PAD - N_EDGES
    src = jnp.concatenate(
        [edge_index[0].astype(jnp.int32), jnp.zeros((npad,), jnp.int32)]
    ).reshape(ND, NW, NCHUNK, CHUNK)
    pad_dst = N_NODES + jnp.arange(npad, dtype=jnp.int32) % (NP - N_NODES)
    dst = jnp.concatenate(
        [edge_index[1].astype(jnp.int32), pad_dst]
    ).reshape(ND, NW, NCHUNK, CHUNK)
    hP = jnp.zeros((NP, D), jnp.float32).at[:N_NODES].set(x)
    zrow = jnp.zeros((RPS, D), jnp.float32)
    ones = jnp.ones((CHUNK, CNT_W), jnp.float32)
    dummy = jnp.zeros((1, D), jnp.float32)
    r2 = lambda v: v.reshape(1, D)

    devs = jax.devices()[:ND]
    mesh = Mesh(np.array(devs), ("d",))
    rep = P()
    fn = jax.shard_map(
        _pipeline, mesh=mesh,
        in_specs=(rep, P("d"), P("d")) + (rep,) * 16,
        out_specs=rep,
        check_vma=False,
    )
    h3P = fn(hP, src, dst, zrow, ones,
             W_rel1.T, r2(b_rel1), W_root1.T, r2(gamma1), r2(beta1),
             W_rel2.T, r2(b_rel2), W_root2.T, r2(gamma2), r2(beta2),
             W_rel3.T, r2(b_rel3), W_root3.T, dummy)
    return h3P[:N_NODES]


# speedup vs baseline: 2.0880x; 2.0880x over previous
import functools

import jax
import jax.numpy as jnp
import numpy as np
from jax import lax
from jax.experimental import pallas as pl
from jax.experimental.pallas import tpu as pltpu
from jax.experimental.pallas import tpu_sc as plsc
from jax.sharding import Mesh, PartitionSpec as P

N_NODES = 10000
N_EDGES = 320000
D = 128
ND = 2
NC = 2
NS = 16
NW = NC * NS
CHUNK = 128
EPW = 5120
E_PAD = ND * NW * EPW
NCHUNK = EPW // CHUNK
IBLK = 8
NBLK = NCHUNK // IBLK
DEPTH = 2
NP = 10240
RPS = NP // NS
HND = NP // ND
CNT_W = 128
BN_SCALE = 1.0 / np.sqrt(1.0 + 1e-5)


def _seg_sum_body(h_hbm, src_hbm, dst_hbm, zrow_hbm, agg_out,
                  srcv, dstv, rows, sems, aggsh):
    c = lax.axis_index("c")
    s = lax.axis_index("s")
    wid = c * NS + s

    pltpu.sync_copy(zrow_hbm, aggsh.at[pl.ds(s * RPS, RPS)])
    plsc.subcore_barrier()

    def start_gather(ci, k):
        pltpu.async_copy(h_hbm.at[srcv.at[ci]], rows[k], sems[k])

    def wait_gather(ci, k):
        pltpu.make_async_copy(h_hbm.at[srcv.at[ci]], rows[k], sems[k]).wait()

    def scatter_add(ci, k):
        pltpu.sync_copy(rows[k], aggsh.at[dstv.at[ci]], add=True)

    @pl.loop(0, NBLK)
    def _(blk):
        pltpu.sync_copy(src_hbm.at[wid].at[pl.ds(blk * IBLK, IBLK)], srcv)
        pltpu.sync_copy(dst_hbm.at[wid].at[pl.ds(blk * IBLK, IBLK)], dstv)
        for k in range(DEPTH):
            start_gather(k, k)

        @pl.loop(0, IBLK - DEPTH, step=DEPTH)
        def _(ci):
            for k in range(DEPTH):
                wait_gather(ci + k, k)
                scatter_add(ci + k, k)
                start_gather(ci + k + DEPTH, k)

        for k in range(DEPTH):
            wait_gather(IBLK - DEPTH + k, k)
            scatter_add(IBLK - DEPTH + k, k)

    plsc.subcore_barrier()
    pltpu.sync_copy(aggsh.at[pl.ds(s * RPS, RPS)],
                    agg_out.at[c].at[pl.ds(s * RPS, RPS)])


_seg_sum = pl.kernel(
    _seg_sum_body,
    out_type=jax.ShapeDtypeStruct((NC, NP, D), jnp.float32),
    mesh=plsc.VectorSubcoreMesh(core_axis_name="c", subcore_axis_name="s"),
    scratch_types=[
        pltpu.VMEM((IBLK, CHUNK), jnp.int32),
        pltpu.VMEM((IBLK, CHUNK), jnp.int32),
        [pltpu.VMEM((CHUNK, D), jnp.float32) for _ in range(DEPTH)],
        [pltpu.SemaphoreType.DMA for _ in range(DEPTH)],
        pltpu.VMEM_SHARED((NP, D), jnp.float32),
    ],
    name="seg_sum",
)


def _cnt_body(dst_hbm, zcnt_hbm, ones_hbm, cnt_out, dstv, onesv, cntsh):
    c = lax.axis_index("c")
    s = lax.axis_index("s")
    wid = c * NS + s

    pltpu.sync_copy(zcnt_hbm, cntsh.at[pl.ds(s * RPS, RPS)])
    pltpu.sync_copy(ones_hbm, onesv)
    plsc.subcore_barrier()

    @pl.loop(0, NBLK)
    def _(blk):
        pltpu.sync_copy(dst_hbm.at[wid].at[pl.ds(blk * IBLK, IBLK)], dstv)

        @pl.loop(0, IBLK)
        def _(ci):
            pltpu.sync_copy(onesv, cntsh.at[dstv.at[ci]], add=True)

    plsc.subcore_barrier()
    pltpu.sync_copy(cntsh.at[pl.ds(s * RPS, RPS)],
                    cnt_out.at[c].at[pl.ds(s * RPS, RPS)])


_seg_cnt = pl.kernel(
    _cnt_body,
    out_type=jax.ShapeDtypeStruct((NC, NP, CNT_W), jnp.float32),
    mesh=plsc.VectorSubcoreMesh(core_axis_name="c", subcore_axis_name="s"),
    scratch_types=[
        pltpu.VMEM((IBLK, CHUNK), jnp.int32),
        pltpu.VMEM((CHUNK, CNT_W), jnp.float32),
        pltpu.VMEM_SHARED((NP, CNT_W), jnp.float32),
    ],
    name="seg_cnt",
)


def _tc_layer_body(relu, h_ref, pa_ref, inv_ref, wr_ref, b_ref,
                   wt_ref, g_ref, be_ref, o_ref):
    agg = (pa_ref[0] + pa_ref[1]) * inv_ref[:, 0:1]
    out = (jnp.dot(agg, wr_ref[...], preferred_element_type=jnp.float32)
           + b_ref[...]
           + jnp.dot(h_ref[...], wt_ref[...], preferred_element_type=jnp.float32))
    if relu:
        out = jnp.maximum(out * (g_ref[...] * BN_SCALE) + be_ref[...], 0.0)
    o_ref[...] = out


def _make_tc_layer(relu, rows=HND, block_rows=1280):
    grid = (rows // block_rows,)
    return pl.pallas_call(
        functools.partial(_tc_layer_body, relu),
        grid=grid,
        in_specs=[
            pl.BlockSpec((block_rows, D), lambda i: (i, 0)),
            pl.BlockSpec((ND, block_rows, D), lambda i: (0, i, 0)),
            pl.BlockSpec((block_rows, CNT_W), lambda i: (i, 0)),
            pl.BlockSpec((D, D), lambda i: (0, 0)),
            pl.BlockSpec((1, D), lambda i: (0, 0)),
            pl.BlockSpec((D, D), lambda i: (0, 0)),
            pl.BlockSpec((1, D), lambda i: (0, 0)),
            pl.BlockSpec((1, D), lambda i: (0, 0)),
        ],
        out_specs=pl.BlockSpec((block_rows, D), lambda i: (i, 0)),
        out_shape=jax.ShapeDtypeStruct((rows, D), jnp.float32),
        name="gconv_dense_relu" if relu else "gconv_dense",
    )


_tc_layer_relu = _make_tc_layer(True)
_tc_layer_plain = _make_tc_layer(False)


def _combine_body(pa_ref, o_ref):
    o_ref[...] = pa_ref[0] + pa_ref[1]


_tc_combine = pl.pallas_call(
    _combine_body,
    grid=(NP // 1280,),
    in_specs=[pl.BlockSpec((NC, 1280, D), lambda i: (0, i, 0))],
    out_specs=pl.BlockSpec((1280, D), lambda i: (i, 0)),
    out_shape=jax.ShapeDtypeStruct((NP, D), jnp.float32),
    name="combine_partials",
)


def _inv_body(c_ref, o_ref):
    o_ref[...] = 1.0 / jnp.maximum(c_ref[0] + c_ref[1], 1.0)


_tc_inv = pl.pallas_call(
    _inv_body,
    grid=(HND // 1280,),
    in_specs=[pl.BlockSpec((ND, 1280, CNT_W), lambda i: (0, i, 0))],
    out_specs=pl.BlockSpec((1280, CNT_W), lambda i: (i, 0)),
    out_shape=jax.ShapeDtypeStruct((HND, CNT_W), jnp.float32),
    name="inv_counts",
)


def _pipeline(hP, src, dst, zrow, ones, WrT1, b1, WtT1, g1, be1,
              WrT2, b2, WtT2, g2, be2, WrT3, b3, WtT3, dummy):
    d = lax.axis_index("d")
    src = src.reshape(NW, NCHUNK, CHUNK)
    dst = dst.reshape(NW, NCHUNK, CHUNK)

    def local_rows(a):
        return lax.dynamic_slice_in_dim(a, d * HND, HND, axis=0)

    def gather_planes(tot):
        g = lax.all_gather(tot, "d", axis=0, tiled=False)
        return lax.dynamic_slice_in_dim(g, d * HND, HND, axis=1)

    cnts = _seg_cnt(dst, zrow, ones)
    invL = _tc_inv(gather_planes(_tc_combine(cnts)))

    def layer(hP, tc, *w):
        agg = _seg_sum(hP, src, dst, zrow)
        pa = gather_planes(_tc_combine(agg))
        return tc(local_rows(hP), pa, invL, *w)

    h1 = layer(hP, _tc_layer_relu, WrT1, b1, WtT1, g1, be1)
    h1P = lax.all_gather(h1, "d", axis=0, tiled=True)
    h2 = layer(h1P, _tc_layer_relu, WrT2, b2, WtT2, g2, be2)
    h2P = lax.all_gather(h2, "d", axis=0, tiled=True)
    h3 = layer(h2P, _tc_layer_plain, WrT3, b3, WtT3, dummy, dummy)
    return lax.all_gather(h3, "d", axis=0, tiled=True)


def kernel(x, edge_index, W_rel1, b_rel1, W_root1, W_rel2, b_rel2, W_root2,
           W_rel3, b_rel3, W_root3, gamma1, beta1, gamma2, beta2):
    npad = E_PAD - N_EDGES
    nworkers = ND * NW
    rpw = N_EDGES // nworkers
    ppw = npad // nworkers
    pad_iota = jnp.arange(npad, dtype=jnp.int32)

    def per_worker(idx, pad):
        a = idx.astype(jnp.int32).reshape(nworkers, rpw)
        b = pad.reshape(nworkers, ppw)
        return jnp.concatenate([a, b], axis=1).reshape(
            ND, NW, NCHUNK, CHUNK)

    src = per_worker(edge_index[0], pad_iota % N_NODES)
    dst = per_worker(edge_index[1], N_NODES + pad_iota % (NP - N_NODES))
    hP = jnp.zeros((NP, D), jnp.float32).at[:N_NODES].set(x)
    zrow = jnp.zeros((RPS, D), jnp.float32)
    ones = jnp.ones((CHUNK, CNT_W), jnp.float32)
    dummy = jnp.zeros((1, D), jnp.float32)
    r2 = lambda v: v.reshape(1, D)

    devs = jax.devices()[:ND]
    mesh = Mesh(np.array(devs), ("d",))
    rep = P()
    fn = jax.shard_map(
        _pipeline, mesh=mesh,
        in_specs=(rep, P("d"), P("d")) + (rep,) * 16,
        out_specs=rep,
        check_vma=False,
    )
    h3P = fn(hP, src, dst, zrow, ones,
             W_rel1.T, r2(b_rel1), W_root1.T, r2(gamma1), r2(beta1),
             W_rel2.T, r2(b_rel2), W_root2.T, r2(gamma2), r2(beta2),
             W_rel3.T, r2(b_rel3), W_root3.T, dummy)
    return h3P[:N_NODES]
